# baseline (speedup 1.0000x reference)
import functools
import jax
import jax.numpy as jnp
from jax import lax
from jax.experimental import pallas as pl
from jax.experimental.pallas import tpu as pltpu
from jax.experimental.pallas import tpu_sc as plsc

N = 10000
E = 320000
IN_DIM = 512
NDIM = 128
K = 4
DD = NDIM // K
ROUTIT = 6
NLAYER = 4
NCLASS = 40

NC = 2
NS = 16
NW = NC * NS
EW = E // NW
B = 16
NCHUNK = EW // B
NG = B // 16
RPT = 624
RTAIL = N - RPT * NS

MBLK = 1000
GRID = N // MBLK


def _route_body(xn_hbm, cn_hbm, src_hbm, trg_hbm, zer_hbm, out_hbm,
                z0, c0, z1, c1, src_i, trg_i, acc, s0, s1, s2, s3):
    cid = lax.axis_index("c")
    sid = lax.axis_index("s")
    wid = sid * NC + cid

    pltpu.sync_copy(zer_hbm.at[pl.ds(sid * RPT, RPT)],
                    acc.at[pl.ds(sid * RPT, RPT)])

    @pl.when(sid == NS - 1)
    def _():
        pltpu.sync_copy(zer_hbm.at[pl.ds(RPT * NS, RTAIL)],
                        acc.at[pl.ds(RPT * NS, RTAIL)])

    plsc.subcore_barrier()

    pltpu.sync_copy(src_hbm.at[pl.ds(wid * EW, EW)], src_i)
    pltpu.sync_copy(trg_hbm.at[pl.ds(wid * EW, EW)], trg_i)

    slots = ((z0, c0, s0, s1), (z1, c1, s2, s3))

    def fire(ci, slot):
        z, c, sz, sc = slot
        sv = src_i[pl.ds(ci * B, B)]
        tv = trg_i[pl.ds(ci * B, B)]
        pltpu.async_copy(xn_hbm.at[sv], z, sz)
        pltpu.async_copy(cn_hbm.at[tv], c, sc)

    def process(ci, slot):
        z, c, sz, sc = slot
        sv = src_i[pl.ds(ci * B, B)]
        tv = trg_i[pl.ds(ci * B, B)]
        pltpu.make_async_copy(xn_hbm.at[sv], z, sz).wait()
        pltpu.make_async_copy(cn_hbm.at[tv], c, sc).wait()

        def group_body(g, gcarry):
            rows = g * 16 + lax.iota(jnp.int32, 16)
            ps = []
            for k in range(K):
                pk = jnp.zeros((16,), jnp.float32)
                for t in range(DD):
                    col = jnp.full((16,), k * DD + t, jnp.int32)
                    zt = plsc.load_gather(z, [rows, col])
                    ct = plsc.load_gather(c, [rows, col])
                    pk = pk + zt * ct
                ps.append(pk)
            es = [jnp.exp(p) for p in ps]
            ssum = (es[0] + es[1]) + (es[2] + es[3])
            ws = [e / ssum for e in es]
            for k in range(K):
                for t in range(DD):
                    col = jnp.full((16,), k * DD + t, jnp.int32)
                    zt = plsc.load_gather(z, [rows, col])
                    plsc.store_scatter(c, [rows, col], zt * ws[k])
            return gcarry

        lax.fori_loop(0, NG, group_body, 0)
        pltpu.sync_copy(c, acc.at[tv], add=True)

    fire(0, slots[0])
    fire(1, slots[1])

    def pair_body(i, carry):
        for b in range(2):
            ci = i * 2 + b
            process(ci, slots[b])

            @pl.when(ci + 2 < NCHUNK)
            def _():
                fire(ci + 2, slots[b])

        return carry

    lax.fori_loop(0, NCHUNK // 2, pair_body, 0)
    process(NCHUNK - 1, slots[0])
    plsc.subcore_barrier()
    pltpu.sync_copy(acc.at[pl.ds(sid * RPT, RPT)],
                    out_hbm.at[cid, pl.ds(sid * RPT, RPT)])

    @pl.when(sid == NS - 1)
    def _():
        pltpu.sync_copy(acc.at[pl.ds(RPT * NS, RTAIL)],
                        out_hbm.at[cid, pl.ds(RPT * NS, RTAIL)])


_route = pl.kernel(
    _route_body,
    out_type=jax.ShapeDtypeStruct((NC, N, NDIM), jnp.float32),
    mesh=plsc.VectorSubcoreMesh(core_axis_name="c", subcore_axis_name="s"),
    compiler_params=pltpu.CompilerParams(needs_layout_passes=False),
    scratch_types=[
        pltpu.VMEM((B, NDIM), jnp.float32),
        pltpu.VMEM((B, NDIM), jnp.float32),
        pltpu.VMEM((B, NDIM), jnp.float32),
        pltpu.VMEM((B, NDIM), jnp.float32),
        pltpu.VMEM((EW,), jnp.int32),
        pltpu.VMEM((EW,), jnp.int32),
        pltpu.VMEM_SHARED((N, NDIM), jnp.float32),
        pltpu.SemaphoreType.DMA,
        pltpu.SemaphoreType.DMA,
        pltpu.SemaphoreType.DMA,
        pltpu.SemaphoreType.DMA,
    ],
)


def _capsule_norm(c, mask):
    s = jnp.dot(c * c, mask, preferred_element_type=jnp.float32)
    return c / jnp.maximum(jnp.sqrt(s), 1e-12)


def _pca_body(feat_ref, w_ref, b_ref, mask_ref, o_ref):
    x = jnp.dot(feat_ref[...], w_ref[...], preferred_element_type=jnp.float32)
    x = jnp.maximum(x + b_ref[...], 0.0)
    o_ref[...] = _capsule_norm(x, mask_ref[...])


def _comb_norm_body(c_ref, p0_ref, p1_ref, mask_ref, o_ref):
    c = c_ref[...] + p0_ref[...] + p1_ref[...]
    o_ref[...] = _capsule_norm(c, mask_ref[...])


def _comb_relu_norm_body(c_ref, p0_ref, p1_ref, mask_ref, o_ref):
    c = jnp.maximum(c_ref[...] + p0_ref[...] + p1_ref[...], 0.0)
    o_ref[...] = _capsule_norm(c, mask_ref[...])


def _comb_relu_body(c_ref, p0_ref, p1_ref, o_ref):
    o_ref[...] = jnp.maximum(c_ref[...] + p0_ref[...] + p1_ref[...], 0.0)


def _mlp_body(x_ref, w_ref, b_ref, o_ref):
    logits = jnp.dot(x_ref[...], w_ref[...],
                     preferred_element_type=jnp.float32) + b_ref[...]
    valid = lax.broadcasted_iota(jnp.int32, logits.shape, 1) < NCLASS
    masked = jnp.where(valid, logits, -1e30)
    mx = jnp.max(masked, axis=1, keepdims=True)
    sh = masked - mx
    lse = jnp.log(jnp.sum(jnp.where(valid, jnp.exp(sh), 0.0), axis=1,
                          keepdims=True))
    o_ref[...] = sh - lse


def _row_call(body, full_shapes):
    def make(blocked_cols, out_cols=NDIM):
        in_specs = [pl.BlockSpec((MBLK, c), lambda i: (i, 0))
                    for c in blocked_cols]
        in_specs += [pl.BlockSpec(fs, lambda i: (0, 0)) for fs in full_shapes]
        return pl.pallas_call(
            body,
            grid=(GRID,),
            in_specs=in_specs,
            out_specs=pl.BlockSpec((MBLK, out_cols), lambda i: (i, 0)),
            out_shape=jax.ShapeDtypeStruct((N, out_cols), jnp.float32),
        )
    return make


_pca = _row_call(_pca_body, [(IN_DIM, NDIM), (1, NDIM), (NDIM, NDIM)])([IN_DIM])
_comb_norm = _row_call(_comb_norm_body, [(NDIM, NDIM)])([NDIM, NDIM, NDIM])
_comb_relu_norm = _row_call(_comb_relu_norm_body, [(NDIM, NDIM)])(
    [NDIM, NDIM, NDIM])
_comb_relu = _row_call(_comb_relu_body, [])([NDIM, NDIM, NDIM])
_mlp = _row_call(_mlp_body, [(NDIM, NDIM), (1, NDIM)])([NDIM])


@jax.jit
def kernel(feat, src_trg_edges, W_pca, b_pca, W_mlp, b_mlp):
    src = src_trg_edges[0]
    trg = src_trg_edges[1]
    caps_mask = jnp.kron(jnp.eye(K, dtype=jnp.float32),
                         jnp.ones((DD, DD), jnp.float32))
    zeros_n = jnp.zeros((N, NDIM), jnp.float32)

    xn = _pca(feat, W_pca, b_pca.reshape(1, NDIM), caps_mask)
    for layer in range(NLAYER):
        cn = xn
        for t in range(ROUTIT):
            part = _route(xn, cn, src, trg, zeros_n)
            if t < ROUTIT - 1:
                cn = _comb_norm(cn, part[0], part[1], caps_mask)
            elif layer < NLAYER - 1:
                xn = _comb_relu_norm(cn, part[0], part[1], caps_mask)
            else:
                x_out = _comb_relu(cn, part[0], part[1])

    w_pad = jnp.zeros((NDIM, NDIM), jnp.float32).at[:, :NCLASS].set(W_mlp)
    b_pad = jnp.zeros((1, NDIM), jnp.float32).at[0, :NCLASS].set(b_mlp)
    out = _mlp(x_out, w_pad, b_pad)
    return out[:, :NCLASS]


# speedup vs baseline: 1.0801x; 1.0801x over previous
import functools
import jax
import jax.numpy as jnp
from jax import lax
from jax.experimental import pallas as pl
from jax.experimental.pallas import tpu as pltpu
from jax.experimental.pallas import tpu_sc as plsc

N = 10000
E = 320000
IN_DIM = 512
NDIM = 128
K = 4
DD = NDIM // K
ROUTIT = 6
NLAYER = 4
NCLASS = 40

NC = 2
NS = 16
NW = NC * NS
EW = E // NW
B = 80
NCHUNK = EW // B
NG = B // 16
RPT = 624
RTAIL = N - RPT * NS

MBLK = 1000
GRID = N // MBLK


def _route_body(xn_hbm, cn_hbm, src_hbm, trg_hbm, zer_hbm, out_hbm,
                z0, z1, c0, si0, ti0, si1, ti1, acc, s0, s1, s2):
    cid = lax.axis_index("c")
    sid = lax.axis_index("s")
    wid = sid * NC + cid

    pltpu.sync_copy(zer_hbm.at[pl.ds(sid * RPT, RPT)],
                    acc.at[pl.ds(sid * RPT, RPT)])

    @pl.when(sid == NS - 1)
    def _():
        pltpu.sync_copy(zer_hbm.at[pl.ds(RPT * NS, RTAIL)],
                        acc.at[pl.ds(RPT * NS, RTAIL)])

    plsc.subcore_barrier()

    slots = ((z0, si0, ti0, s0), (z1, si1, ti1, s1))
    c = c0
    sc = s2

    def fire(ci, slot):
        z, si, ti, sz = slot
        ebase = wid * EW + ci * B
        pltpu.sync_copy(src_hbm.at[pl.ds(ebase, B)], si)
        pltpu.sync_copy(trg_hbm.at[pl.ds(ebase, B)], ti)
        pltpu.async_copy(xn_hbm.at[si], z, sz)

    def process(ci, slot):
        z, si, ti, sz = slot
        pltpu.async_copy(cn_hbm.at[ti], c, sc).wait()
        pltpu.make_async_copy(xn_hbm.at[si], z, sz).wait()

        def group_body(g, gcarry):
            rows = g * 16 + lax.iota(jnp.int32, 16)
            ps = []
            for k in range(K):
                pk = jnp.zeros((16,), jnp.float32)
                for t in range(DD):
                    col = jnp.full((16,), k * DD + t, jnp.int32)
                    zt = plsc.load_gather(z, [rows, col])
                    ct = plsc.load_gather(c, [rows, col])
                    pk = pk + zt * ct
                ps.append(pk)
            es = [jnp.exp(p) for p in ps]
            ssum = (es[0] + es[1]) + (es[2] + es[3])
            ws = [e / ssum for e in es]
            for k in range(K):
                for t in range(DD):
                    col = jnp.full((16,), k * DD + t, jnp.int32)
                    zt = plsc.load_gather(z, [rows, col])
                    plsc.store_scatter(c, [rows, col], zt * ws[k])
            return gcarry

        lax.fori_loop(0, NG, group_body, 0)
        pltpu.sync_copy(c, acc.at[ti], add=True)

    fire(0, slots[0])
    fire(1, slots[1])

    def pair_body(i, carry):
        for b in range(2):
            ci = i * 2 + b
            process(ci, slots[b])

            @pl.when(ci + 2 < NCHUNK)
            def _():
                fire(ci + 2, slots[b])

        return carry

    lax.fori_loop(0, NCHUNK // 2, pair_body, 0)
    process(NCHUNK - 1, slots[0])
    plsc.subcore_barrier()
    pltpu.sync_copy(acc.at[pl.ds(sid * RPT, RPT)],
                    out_hbm.at[cid, pl.ds(sid * RPT, RPT)])

    @pl.when(sid == NS - 1)
    def _():
        pltpu.sync_copy(acc.at[pl.ds(RPT * NS, RTAIL)],
                        out_hbm.at[cid, pl.ds(RPT * NS, RTAIL)])


_route = pl.kernel(
    _route_body,
    out_type=jax.ShapeDtypeStruct((NC, N, NDIM), jnp.float32),
    mesh=plsc.VectorSubcoreMesh(core_axis_name="c", subcore_axis_name="s"),
    compiler_params=pltpu.CompilerParams(needs_layout_passes=False),
    scratch_types=[
        pltpu.VMEM((B, NDIM), jnp.float32),
        pltpu.VMEM((B, NDIM), jnp.float32),
        pltpu.VMEM((B, NDIM), jnp.float32),
        pltpu.VMEM((B,), jnp.int32),
        pltpu.VMEM((B,), jnp.int32),
        pltpu.VMEM((B,), jnp.int32),
        pltpu.VMEM((B,), jnp.int32),
        pltpu.VMEM_SHARED((N, NDIM), jnp.float32),
        pltpu.SemaphoreType.DMA,
        pltpu.SemaphoreType.DMA,
        pltpu.SemaphoreType.DMA,
    ],
)


def _capsule_norm(c, mask):
    s = jnp.dot(c * c, mask, preferred_element_type=jnp.float32)
    return c / jnp.maximum(jnp.sqrt(s), 1e-12)


def _pca_body(feat_ref, w_ref, b_ref, mask_ref, o_ref):
    x = jnp.dot(feat_ref[...], w_ref[...], preferred_element_type=jnp.float32)
    x = jnp.maximum(x + b_ref[...], 0.0)
    o_ref[...] = _capsule_norm(x, mask_ref[...])


def _comb_norm_body(c_ref, p0_ref, p1_ref, mask_ref, o_ref):
    c = c_ref[...] + p0_ref[...] + p1_ref[...]
    o_ref[...] = _capsule_norm(c, mask_ref[...])


def _comb_relu_norm_body(c_ref, p0_ref, p1_ref, mask_ref, o_ref):
    c = jnp.maximum(c_ref[...] + p0_ref[...] + p1_ref[...], 0.0)
    o_ref[...] = _capsule_norm(c, mask_ref[...])


def _comb_relu_body(c_ref, p0_ref, p1_ref, o_ref):
    o_ref[...] = jnp.maximum(c_ref[...] + p0_ref[...] + p1_ref[...], 0.0)


def _mlp_body(x_ref, w_ref, b_ref, o_ref):
    logits = jnp.dot(x_ref[...], w_ref[...],
                     preferred_element_type=jnp.float32) + b_ref[...]
    valid = lax.broadcasted_iota(jnp.int32, logits.shape, 1) < NCLASS
    masked = jnp.where(valid, logits, -1e30)
    mx = jnp.max(masked, axis=1, keepdims=True)
    sh = masked - mx
    lse = jnp.log(jnp.sum(jnp.where(valid, jnp.exp(sh), 0.0), axis=1,
                          keepdims=True))
    o_ref[...] = sh - lse


def _row_call(body, full_shapes):
    def make(blocked_cols, out_cols=NDIM):
        in_specs = [pl.BlockSpec((MBLK, c), lambda i: (i, 0))
                    for c in blocked_cols]
        in_specs += [pl.BlockSpec(fs, lambda i: (0, 0)) for fs in full_shapes]
        return pl.pallas_call(
            body,
            grid=(GRID,),
            in_specs=in_specs,
            out_specs=pl.BlockSpec((MBLK, out_cols), lambda i: (i, 0)),
            out_shape=jax.ShapeDtypeStruct((N, out_cols), jnp.float32),
        )
    return make


_pca = _row_call(_pca_body, [(IN_DIM, NDIM), (1, NDIM), (NDIM, NDIM)])([IN_DIM])
_comb_norm = _row_call(_comb_norm_body, [(NDIM, NDIM)])([NDIM, NDIM, NDIM])
_comb_relu_norm = _row_call(_comb_relu_norm_body, [(NDIM, NDIM)])(
    [NDIM, NDIM, NDIM])
_comb_relu = _row_call(_comb_relu_body, [])([NDIM, NDIM, NDIM])
_mlp = _row_call(_mlp_body, [(NDIM, NDIM), (1, NDIM)])([NDIM])


@jax.jit
def kernel(feat, src_trg_edges, W_pca, b_pca, W_mlp, b_mlp):
    src = src_trg_edges[0]
    trg = src_trg_edges[1]
    caps_mask = jnp.kron(jnp.eye(K, dtype=jnp.float32),
                         jnp.ones((DD, DD), jnp.float32))
    zeros_n = jnp.zeros((N, NDIM), jnp.float32)

    xn = _pca(feat, W_pca, b_pca.reshape(1, NDIM), caps_mask)
    for layer in range(NLAYER):
        cn = xn
        for t in range(ROUTIT):
            part = _route(xn, cn, src, trg, zeros_n)
            if t < ROUTIT - 1:
                cn = _comb_norm(cn, part[0], part[1], caps_mask)
            elif layer < NLAYER - 1:
                xn = _comb_relu_norm(cn, part[0], part[1], caps_mask)
            else:
                x_out = _comb_relu(cn, part[0], part[1])

    w_pad = jnp.zeros((NDIM, NDIM), jnp.float32).at[:, :NCLASS].set(W_mlp)
    b_pad = jnp.zeros((1, NDIM), jnp.float32).at[0, :NCLASS].set(b_mlp)
    out = _mlp(x_out, w_pad, b_pad)
    return out[:, :NCLASS]


# speedup vs baseline: 1.1501x; 1.0648x over previous
import functools
import jax
import jax.numpy as jnp
from jax import lax
from jax.experimental import pallas as pl
from jax.experimental.pallas import tpu as pltpu
from jax.experimental.pallas import tpu_sc as plsc

N = 10000
E = 320000
IN_DIM = 512
NDIM = 128
K = 4
DD = NDIM // K
ROUTIT = 6
NLAYER = 4
NCLASS = 40

NC = 2
NS = 16
NW = NC * NS
EW = E // NW
B = 64
NCHUNK = 156
TAIL = EW - NCHUNK * B
NG = B // 16
RPT = 624
RTAIL = N - RPT * NS

MBLK = 1000
GRID = N // MBLK


def _route_body(xn_hbm, cn_hbm, src_hbm, trg_hbm, zer_hbm, out_hbm,
                z0, z1, c0, c1, si0, ti0, si1, ti1, acc,
                sz0, sz1, sc0, sc1):
    cid = lax.axis_index("c")
    sid = lax.axis_index("s")
    wid = sid * NC + cid

    pltpu.sync_copy(zer_hbm.at[pl.ds(sid * RPT, RPT)],
                    acc.at[pl.ds(sid * RPT, RPT)])

    @pl.when(sid == NS - 1)
    def _():
        pltpu.sync_copy(zer_hbm.at[pl.ds(RPT * NS, RTAIL)],
                        acc.at[pl.ds(RPT * NS, RTAIL)])

    plsc.subcore_barrier()

    slots = ((z0, c0, si0, ti0, sz0, sc0), (z1, c1, si1, ti1, sz1, sc1))

    def fire(ci, slot):
        z, c, si, ti, sz, sc = slot
        ebase = wid * EW + ci * B
        pltpu.sync_copy(src_hbm.at[pl.ds(ebase, B)], si)
        pltpu.sync_copy(trg_hbm.at[pl.ds(ebase, B)], ti)
        pltpu.async_copy(xn_hbm.at[si], z, sz)
        pltpu.async_copy(cn_hbm.at[ti], c, sc)

    def process(ci, slot):
        z, c, si, ti, sz, sc = slot
        pltpu.make_async_copy(xn_hbm.at[si], z, sz).wait()
        pltpu.make_async_copy(cn_hbm.at[ti], c, sc).wait()

        def group_body(g, gcarry):
            rows = g * 16 + lax.iota(jnp.int32, 16)
            ps = []
            for k in range(K):
                pk = jnp.zeros((16,), jnp.float32)
                for t in range(DD):
                    col = jnp.full((16,), k * DD + t, jnp.int32)
                    zt = plsc.load_gather(z, [rows, col])
                    ct = plsc.load_gather(c, [rows, col])
                    pk = pk + zt * ct
                ps.append(pk)
            es = [jnp.exp(p) for p in ps]
            ssum = (es[0] + es[1]) + (es[2] + es[3])
            ws = [e / ssum for e in es]
            for k in range(K):
                for t in range(DD):
                    col = jnp.full((16,), k * DD + t, jnp.int32)
                    zt = plsc.load_gather(z, [rows, col])
                    plsc.store_scatter(c, [rows, col], zt * ws[k])
            return gcarry

        lax.fori_loop(0, NG, group_body, 0)
        pltpu.sync_copy(c, acc.at[ti], add=True)

    fire(0, slots[0])
    fire(1, slots[1])

    def pair_body(i, carry):
        for b in range(2):
            ci = i * 2 + b
            process(ci, slots[b])

            @pl.when(ci + 2 < NCHUNK)
            def _():
                fire(ci + 2, slots[b])

        return carry

    lax.fori_loop(0, NCHUNK // 2, pair_body, 0)

    tbase = wid * EW + NCHUNK * B
    pltpu.sync_copy(src_hbm.at[pl.ds(tbase, TAIL)], si0.at[pl.ds(0, TAIL)])
    pltpu.sync_copy(trg_hbm.at[pl.ds(tbase, TAIL)], ti0.at[pl.ds(0, TAIL)])
    sv = si0[pl.ds(0, TAIL)]
    tv = ti0[pl.ds(0, TAIL)]
    pltpu.async_copy(xn_hbm.at[sv], z0.at[pl.ds(0, TAIL)], sz0).wait()
    pltpu.async_copy(cn_hbm.at[tv], c0.at[pl.ds(0, TAIL)], sc0).wait()
    rows = lax.iota(jnp.int32, 16)
    ps = []
    for k in range(K):
        pk = jnp.zeros((16,), jnp.float32)
        for t in range(DD):
            col = jnp.full((16,), k * DD + t, jnp.int32)
            zt = plsc.load_gather(z0, [rows, col])
            ct = plsc.load_gather(c0, [rows, col])
            pk = pk + zt * ct
        ps.append(pk)
    es = [jnp.exp(p) for p in ps]
    ssum = (es[0] + es[1]) + (es[2] + es[3])
    ws = [e / ssum for e in es]
    for k in range(K):
        for t in range(DD):
            col = jnp.full((16,), k * DD + t, jnp.int32)
            zt = plsc.load_gather(z0, [rows, col])
            plsc.store_scatter(c0, [rows, col], zt * ws[k])
    pltpu.sync_copy(c0.at[pl.ds(0, TAIL)], acc.at[tv], add=True)

    plsc.subcore_barrier()
    pltpu.sync_copy(acc.at[pl.ds(sid * RPT, RPT)],
                    out_hbm.at[cid, pl.ds(sid * RPT, RPT)])

    @pl.when(sid == NS - 1)
    def _():
        pltpu.sync_copy(acc.at[pl.ds(RPT * NS, RTAIL)],
                        out_hbm.at[cid, pl.ds(RPT * NS, RTAIL)])


_route = pl.kernel(
    _route_body,
    out_type=jax.ShapeDtypeStruct((NC, N, NDIM), jnp.float32),
    mesh=plsc.VectorSubcoreMesh(core_axis_name="c", subcore_axis_name="s"),
    compiler_params=pltpu.CompilerParams(needs_layout_passes=False),
    scratch_types=[
        pltpu.VMEM((B, NDIM), jnp.float32),
        pltpu.VMEM((B, NDIM), jnp.float32),
        pltpu.VMEM((B, NDIM), jnp.float32),
        pltpu.VMEM((B, NDIM), jnp.float32),
        pltpu.VMEM((B,), jnp.int32),
        pltpu.VMEM((B,), jnp.int32),
        pltpu.VMEM((B,), jnp.int32),
        pltpu.VMEM((B,), jnp.int32),
        pltpu.VMEM_SHARED((N, NDIM), jnp.float32),
        pltpu.SemaphoreType.DMA,
        pltpu.SemaphoreType.DMA,
        pltpu.SemaphoreType.DMA,
        pltpu.SemaphoreType.DMA,
    ],
)


def _capsule_norm(c, mask):
    s = jnp.dot(c * c, mask, preferred_element_type=jnp.float32)
    return c / jnp.maximum(jnp.sqrt(s), 1e-12)


def _pca_body(feat_ref, w_ref, b_ref, mask_ref, o_ref):
    x = jnp.dot(feat_ref[...], w_ref[...], preferred_element_type=jnp.float32)
    x = jnp.maximum(x + b_ref[...], 0.0)
    o_ref[...] = _capsule_norm(x, mask_ref[...])


def _comb_norm_body(c_ref, p0_ref, p1_ref, mask_ref, o_ref):
    c = c_ref[...] + p0_ref[...] + p1_ref[...]
    o_ref[...] = _capsule_norm(c, mask_ref[...])


def _comb_relu_norm_body(c_ref, p0_ref, p1_ref, mask_ref, o_ref):
    c = jnp.maximum(c_ref[...] + p0_ref[...] + p1_ref[...], 0.0)
    o_ref[...] = _capsule_norm(c, mask_ref[...])


def _comb_relu_body(c_ref, p0_ref, p1_ref, o_ref):
    o_ref[...] = jnp.maximum(c_ref[...] + p0_ref[...] + p1_ref[...], 0.0)


def _mlp_body(x_ref, w_ref, b_ref, o_ref):
    logits = jnp.dot(x_ref[...], w_ref[...],
                     preferred_element_type=jnp.float32) + b_ref[...]
    valid = lax.broadcasted_iota(jnp.int32, logits.shape, 1) < NCLASS
    masked = jnp.where(valid, logits, -1e30)
    mx = jnp.max(masked, axis=1, keepdims=True)
    sh = masked - mx
    lse = jnp.log(jnp.sum(jnp.where(valid, jnp.exp(sh), 0.0), axis=1,
                          keepdims=True))
    o_ref[...] = sh - lse


def _row_call(body, full_shapes):
    def make(blocked_cols, out_cols=NDIM):
        in_specs = [pl.BlockSpec((MBLK, c), lambda i: (i, 0))
                    for c in blocked_cols]
        in_specs += [pl.BlockSpec(fs, lambda i: (0, 0)) for fs in full_shapes]
        return pl.pallas_call(
            body,
            grid=(GRID,),
            in_specs=in_specs,
            out_specs=pl.BlockSpec((MBLK, out_cols), lambda i: (i, 0)),
            out_shape=jax.ShapeDtypeStruct((N, out_cols), jnp.float32),
        )
    return make


_pca = _row_call(_pca_body, [(IN_DIM, NDIM), (1, NDIM), (NDIM, NDIM)])([IN_DIM])
_comb_norm = _row_call(_comb_norm_body, [(NDIM, NDIM)])([NDIM, NDIM, NDIM])
_comb_relu_norm = _row_call(_comb_relu_norm_body, [(NDIM, NDIM)])(
    [NDIM, NDIM, NDIM])
_comb_relu = _row_call(_comb_relu_body, [])([NDIM, NDIM, NDIM])
_mlp = _row_call(_mlp_body, [(NDIM, NDIM), (1, NDIM)])([NDIM])


@jax.jit
def kernel(feat, src_trg_edges, W_pca, b_pca, W_mlp, b_mlp):
    src = src_trg_edges[0]
    trg = src_trg_edges[1]
    caps_mask = jnp.kron(jnp.eye(K, dtype=jnp.float32),
                         jnp.ones((DD, DD), jnp.float32))
    zeros_n = jnp.zeros((N, NDIM), jnp.float32)

    xn = _pca(feat, W_pca, b_pca.reshape(1, NDIM), caps_mask)
    for layer in range(NLAYER):
        cn = xn
        for t in range(ROUTIT):
            part = _route(xn, cn, src, trg, zeros_n)
            if t < ROUTIT - 1:
                cn = _comb_norm(cn, part[0], part[1], caps_mask)
            elif layer < NLAYER - 1:
                xn = _comb_relu_norm(cn, part[0], part[1], caps_mask)
            else:
                x_out = _comb_relu(cn, part[0], part[1])

    w_pad = jnp.zeros((NDIM, NDIM), jnp.float32).at[:, :NCLASS].set(W_mlp)
    b_pad = jnp.zeros((1, NDIM), jnp.float32).at[0, :NCLASS].set(b_mlp)
    out = _mlp(x_out, w_pad, b_pad)
    return out[:, :NCLASS]


# speedup vs baseline: 1.2027x; 1.0457x over previous
import functools
import jax
import jax.numpy as jnp
from jax import lax
from jax.experimental import pallas as pl
from jax.experimental.pallas import tpu as pltpu
from jax.experimental.pallas import tpu_sc as plsc

N = 10000
E = 320000
IN_DIM = 512
NDIM = 128
K = 4
DD = NDIM // K
ROUTIT = 6
NLAYER = 4
NCLASS = 40

NC = 2
NS = 16
NW = NC * NS
EW = E // NW
B = 64
NCHUNK = 156
TAIL = EW - NCHUNK * B
NG = B // 16
RPT = 624
RTAIL = N - RPT * NS

MBLK = 1000
GRID = N // MBLK


def _route_body(xn_hbm, cn_hbm, src_hbm, trg_hbm, zer_hbm, out_hbm,
                z0, z1, c0, c1, si0, ti0, si1, ti1, si2, ti2, si3, ti3, acc,
                sz0, sz1, sc0, sc1, sx0, sx1, sx2, sx3):
    cid = lax.axis_index("c")
    sid = lax.axis_index("s")
    wid = sid * NC + cid

    pltpu.sync_copy(zer_hbm.at[pl.ds(sid * RPT, RPT)],
                    acc.at[pl.ds(sid * RPT, RPT)])

    @pl.when(sid == NS - 1)
    def _():
        pltpu.sync_copy(zer_hbm.at[pl.ds(RPT * NS, RTAIL)],
                        acc.at[pl.ds(RPT * NS, RTAIL)])

    plsc.subcore_barrier()

    rslots = ((z0, c0, sz0, sc0), (z1, c1, sz1, sc1))
    islots = ((si0, ti0, sx0), (si1, ti1, sx1), (si2, ti2, sx2),
              (si3, ti3, sx3))

    def fire_idx(ci, islot):
        si, ti, sx = islot
        ebase = wid * EW + ci * B
        pltpu.async_copy(src_hbm.at[pl.ds(ebase, B)], si, sx)
        pltpu.async_copy(trg_hbm.at[pl.ds(ebase, B)], ti, sx)

    def wait_idx(ci, islot):
        si, ti, sx = islot
        ebase = wid * EW + ci * B
        pltpu.make_async_copy(src_hbm.at[pl.ds(ebase, B)], si, sx).wait()
        pltpu.make_async_copy(trg_hbm.at[pl.ds(ebase, B)], ti, sx).wait()

    def fire_rows(islot, rslot):
        si, ti, sx = islot
        z, c, sz, sc = rslot
        pltpu.async_copy(xn_hbm.at[si], z, sz)
        pltpu.async_copy(cn_hbm.at[ti], c, sc)

    def process(ci, islot, rslot):
        si, ti, sx = islot
        z, c, sz, sc = rslot
        pltpu.make_async_copy(xn_hbm.at[si], z, sz).wait()
        pltpu.make_async_copy(cn_hbm.at[ti], c, sc).wait()

        def group_body(g, gcarry):
            rows = g * 16 + lax.iota(jnp.int32, 16)
            ps = []
            for k in range(K):
                pk = jnp.zeros((16,), jnp.float32)
                for t in range(DD):
                    col = jnp.full((16,), k * DD + t, jnp.int32)
                    zt = plsc.load_gather(z, [rows, col])
                    ct = plsc.load_gather(c, [rows, col])
                    pk = pk + zt * ct
                ps.append(pk)
            es = [jnp.exp(p) for p in ps]
            ssum = (es[0] + es[1]) + (es[2] + es[3])
            ws = [e / ssum for e in es]
            for k in range(K):
                for t in range(DD):
                    col = jnp.full((16,), k * DD + t, jnp.int32)
                    zt = plsc.load_gather(z, [rows, col])
                    plsc.store_scatter(c, [rows, col], zt * ws[k])
            return gcarry

        lax.fori_loop(0, NG, group_body, 0)
        pltpu.sync_copy(c, acc.at[ti], add=True)

    for j in range(4):
        fire_idx(j, islots[j])
    for j in range(2):
        wait_idx(j, islots[j])
        fire_rows(islots[j], rslots[j])

    def quad_body(i, carry):
        for b in range(4):
            ci = i * 4 + b
            process(ci, islots[b], rslots[b % 2])

            @pl.when(ci + 4 < NCHUNK)
            def _():
                fire_idx(ci + 4, islots[b])

            @pl.when(ci + 2 < NCHUNK)
            def _():
                wait_idx(ci + 2, islots[(b + 2) % 4])
                fire_rows(islots[(b + 2) % 4], rslots[b % 2])

        return carry

    lax.fori_loop(0, NCHUNK // 4, quad_body, 0)

    tbase = wid * EW + NCHUNK * B
    pltpu.sync_copy(src_hbm.at[pl.ds(tbase, TAIL)], si0.at[pl.ds(0, TAIL)])
    pltpu.sync_copy(trg_hbm.at[pl.ds(tbase, TAIL)], ti0.at[pl.ds(0, TAIL)])
    sv = si0[pl.ds(0, TAIL)]
    tv = ti0[pl.ds(0, TAIL)]
    pltpu.async_copy(xn_hbm.at[sv], z0.at[pl.ds(0, TAIL)], sz0).wait()
    pltpu.async_copy(cn_hbm.at[tv], c0.at[pl.ds(0, TAIL)], sc0).wait()
    rows = lax.iota(jnp.int32, 16)
    ps = []
    for k in range(K):
        pk = jnp.zeros((16,), jnp.float32)
        for t in range(DD):
            col = jnp.full((16,), k * DD + t, jnp.int32)
            zt = plsc.load_gather(z0, [rows, col])
            ct = plsc.load_gather(c0, [rows, col])
            pk = pk + zt * ct
        ps.append(pk)
    es = [jnp.exp(p) for p in ps]
    ssum = (es[0] + es[1]) + (es[2] + es[3])
    ws = [e / ssum for e in es]
    for k in range(K):
        for t in range(DD):
            col = jnp.full((16,), k * DD + t, jnp.int32)
            zt = plsc.load_gather(z0, [rows, col])
            plsc.store_scatter(c0, [rows, col], zt * ws[k])
    pltpu.sync_copy(c0.at[pl.ds(0, TAIL)], acc.at[tv], add=True)

    plsc.subcore_barrier()
    pltpu.sync_copy(acc.at[pl.ds(sid * RPT, RPT)],
                    out_hbm.at[cid, pl.ds(sid * RPT, RPT)])

    @pl.when(sid == NS - 1)
    def _():
        pltpu.sync_copy(acc.at[pl.ds(RPT * NS, RTAIL)],
                        out_hbm.at[cid, pl.ds(RPT * NS, RTAIL)])


_route = pl.kernel(
    _route_body,
    out_type=jax.ShapeDtypeStruct((NC, N, NDIM), jnp.float32),
    mesh=plsc.VectorSubcoreMesh(core_axis_name="c", subcore_axis_name="s"),
    compiler_params=pltpu.CompilerParams(needs_layout_passes=False),
    scratch_types=[
        pltpu.VMEM((B, NDIM), jnp.float32),
        pltpu.VMEM((B, NDIM), jnp.float32),
        pltpu.VMEM((B, NDIM), jnp.float32),
        pltpu.VMEM((B, NDIM), jnp.float32),
        pltpu.VMEM((B,), jnp.int32),
        pltpu.VMEM((B,), jnp.int32),
        pltpu.VMEM((B,), jnp.int32),
        pltpu.VMEM((B,), jnp.int32),
        pltpu.VMEM((B,), jnp.int32),
        pltpu.VMEM((B,), jnp.int32),
        pltpu.VMEM((B,), jnp.int32),
        pltpu.VMEM((B,), jnp.int32),
        pltpu.VMEM_SHARED((N, NDIM), jnp.float32),
        pltpu.SemaphoreType.DMA,
        pltpu.SemaphoreType.DMA,
        pltpu.SemaphoreType.DMA,
        pltpu.SemaphoreType.DMA,
        pltpu.SemaphoreType.DMA,
        pltpu.SemaphoreType.DMA,
        pltpu.SemaphoreType.DMA,
        pltpu.SemaphoreType.DMA,
    ],
)


def _capsule_norm(c, mask):
    s = jnp.dot(c * c, mask, preferred_element_type=jnp.float32)
    return c / jnp.maximum(jnp.sqrt(s), 1e-12)


def _pca_body(feat_ref, w_ref, b_ref, mask_ref, o_ref):
    x = jnp.dot(feat_ref[...], w_ref[...], preferred_element_type=jnp.float32)
    x = jnp.maximum(x + b_ref[...], 0.0)
    o_ref[...] = _capsule_norm(x, mask_ref[...])


def _comb_norm_body(c_ref, p0_ref, p1_ref, mask_ref, o_ref):
    c = c_ref[...] + p0_ref[...] + p1_ref[...]
    o_ref[...] = _capsule_norm(c, mask_ref[...])


def _comb_relu_norm_body(c_ref, p0_ref, p1_ref, mask_ref, o_ref):
    c = jnp.maximum(c_ref[...] + p0_ref[...] + p1_ref[...], 0.0)
    o_ref[...] = _capsule_norm(c, mask_ref[...])


def _comb_relu_body(c_ref, p0_ref, p1_ref, o_ref):
    o_ref[...] = jnp.maximum(c_ref[...] + p0_ref[...] + p1_ref[...], 0.0)


def _mlp_body(x_ref, w_ref, b_ref, o_ref):
    logits = jnp.dot(x_ref[...], w_ref[...],
                     preferred_element_type=jnp.float32) + b_ref[...]
    valid = lax.broadcasted_iota(jnp.int32, logits.shape, 1) < NCLASS
    masked = jnp.where(valid, logits, -1e30)
    mx = jnp.max(masked, axis=1, keepdims=True)
    sh = masked - mx
    lse = jnp.log(jnp.sum(jnp.where(valid, jnp.exp(sh), 0.0), axis=1,
                          keepdims=True))
    o_ref[...] = sh - lse


def _row_call(body, full_shapes):
    def make(blocked_cols, out_cols=NDIM):
        in_specs = [pl.BlockSpec((MBLK, c), lambda i: (i, 0))
                    for c in blocked_cols]
        in_specs += [pl.BlockSpec(fs, lambda i: (0, 0)) for fs in full_shapes]
        return pl.pallas_call(
            body,
            grid=(GRID,),
            in_specs=in_specs,
            out_specs=pl.BlockSpec((MBLK, out_cols), lambda i: (i, 0)),
            out_shape=jax.ShapeDtypeStruct((N, out_cols), jnp.float32),
        )
    return make


_pca = _row_call(_pca_body, [(IN_DIM, NDIM), (1, NDIM), (NDIM, NDIM)])([IN_DIM])
_comb_norm = _row_call(_comb_norm_body, [(NDIM, NDIM)])([NDIM, NDIM, NDIM])
_comb_relu_norm = _row_call(_comb_relu_norm_body, [(NDIM, NDIM)])(
    [NDIM, NDIM, NDIM])
_comb_relu = _row_call(_comb_relu_body, [])([NDIM, NDIM, NDIM])
_mlp = _row_call(_mlp_body, [(NDIM, NDIM), (1, NDIM)])([NDIM])


@jax.jit
def kernel(feat, src_trg_edges, W_pca, b_pca, W_mlp, b_mlp):
    src = src_trg_edges[0]
    trg = src_trg_edges[1]
    caps_mask = jnp.kron(jnp.eye(K, dtype=jnp.float32),
                         jnp.ones((DD, DD), jnp.float32))
    zeros_n = jnp.zeros((N, NDIM), jnp.float32)

    xn = _pca(feat, W_pca, b_pca.reshape(1, NDIM), caps_mask)
    for layer in range(NLAYER):
        cn = xn
        for t in range(ROUTIT):
            part = _route(xn, cn, src, trg, zeros_n)
            if t < ROUTIT - 1:
                cn = _comb_norm(cn, part[0], part[1], caps_mask)
            elif layer < NLAYER - 1:
                xn = _comb_relu_norm(cn, part[0], part[1], caps_mask)
            else:
                x_out = _comb_relu(cn, part[0], part[1])

    w_pad = jnp.zeros((NDIM, NDIM), jnp.float32).at[:, :NCLASS].set(W_mlp)
    b_pad = jnp.zeros((1, NDIM), jnp.float32).at[0, :NCLASS].set(b_mlp)
    out = _mlp(x_out, w_pad, b_pad)
    return out[:, :NCLASS]
